# baseline (speedup 1.0000x reference)
import functools

import jax
import jax.numpy as jnp
from jax import lax
from jax.experimental import pallas as pl
from jax.experimental.pallas import tpu as pltpu
from jax.experimental.pallas import tpu_sc as plsc

B = 4096
L = 200
EMBED = 64
N_ROWS = B * L
NC = 2
NS = 16
NW = NC * NS
ROWS_PER_W = N_ROWS // NW
CHUNK = 512
N_CHUNKS = ROWS_PER_W // CHUNK

_mesh = plsc.VectorSubcoreMesh(
    core_axis_name="c", subcore_axis_name="s", num_cores=NC, num_subcores=NS
)


@functools.partial(
    pl.kernel,
    out_type=jax.ShapeDtypeStruct((N_ROWS, EMBED), jnp.float32),
    mesh=_mesh,
    scratch_types=[
        pltpu.VMEM((CHUNK,), jnp.int32),
        pltpu.VMEM((CHUNK, EMBED), jnp.float32),
        pltpu.SemaphoreType.DMA,
    ],
    compiler_params=pltpu.CompilerParams(use_tc_tiling_on_sc=False),
)
def _gather_kernel(idx_hbm, table_hbm, out_hbm, idx_v, rows_v, sem):
    wid = lax.axis_index("s") * NC + lax.axis_index("c")
    base = wid * ROWS_PER_W

    @pl.loop(0, N_CHUNKS)
    def _chunk_loop(i):
        off = base + i * CHUNK
        pltpu.sync_copy(idx_hbm.at[pl.ds(off, CHUNK)], idx_v)
        pltpu.async_copy(table_hbm.at[idx_v], rows_v, sem).wait()
        pltpu.sync_copy(rows_v, out_hbm.at[pl.ds(off, CHUNK)])


@jax.jit
def kernel(indices, table):
    flat = indices.reshape(-1).astype(jnp.int32)
    out = _gather_kernel(flat, table)
    return out.reshape(indices.shape + (EMBED,))


# speedup vs baseline: 1.0470x; 1.0470x over previous
import functools

import jax
import jax.numpy as jnp
from jax import lax
from jax.experimental import pallas as pl
from jax.experimental.pallas import tpu as pltpu
from jax.experimental.pallas import tpu_sc as plsc

B = 4096
L = 200
EMBED = 64
N_ROWS = B * L
NC = 2
NS = 16
NW = NC * NS
ROWS_PER_W = N_ROWS // NW
CHUNK = 640
N_CHUNKS = ROWS_PER_W // CHUNK

_mesh = plsc.VectorSubcoreMesh(
    core_axis_name="c", subcore_axis_name="s", num_cores=NC, num_subcores=NS
)


@functools.partial(
    pl.kernel,
    out_type=jax.ShapeDtypeStruct((N_ROWS, EMBED), jnp.float32),
    mesh=_mesh,
    scratch_types=[
        pltpu.VMEM((CHUNK,), jnp.int32),
        pltpu.VMEM((CHUNK,), jnp.int32),
        pltpu.VMEM((CHUNK,), jnp.int32),
        pltpu.VMEM((CHUNK,), jnp.int32),
        pltpu.VMEM((CHUNK, EMBED), jnp.float32),
        pltpu.VMEM((CHUNK, EMBED), jnp.float32),
        pltpu.SemaphoreType.DMA,
        pltpu.SemaphoreType.DMA,
        pltpu.SemaphoreType.DMA,
        pltpu.SemaphoreType.DMA,
        pltpu.SemaphoreType.DMA,
        pltpu.SemaphoreType.DMA,
        pltpu.SemaphoreType.DMA,
        pltpu.SemaphoreType.DMA,
    ],
    compiler_params=pltpu.CompilerParams(use_tc_tiling_on_sc=False),
)
def _gather_kernel(idx_hbm, table_hbm, out_hbm, i0, i1, i2, i3, rows0, rows1,
                   si0, si1, si2, si3, g0, g1, s0, s1):
    idxb = (i0, i1, i2, i3)
    isem = (si0, si1, si2, si3)
    rows = (rows0, rows1)
    gsem = (g0, g1)
    ssem = (s0, s1)
    wid = lax.axis_index("s") * NC + lax.axis_index("c")
    base = wid * ROWS_PER_W

    def start_idx(i, m4):
        pltpu.async_copy(idx_hbm.at[wid, i], idxb[m4], isem[m4])

    def start_gather(i, slot, m4):
        pltpu.async_copy(table_hbm.at[idxb[m4]], rows[slot], gsem[slot])

    def start_store(i, slot):
        pltpu.async_copy(
            rows[slot], out_hbm.at[pl.ds(base + i * CHUNK, CHUNK)],
            ssem[slot])

    def drain_idx(m4):
        pltpu.make_async_copy(idx_hbm.at[wid, 0], idxb[m4], isem[m4]).wait()

    def drain_gather(slot):
        pltpu.make_async_copy(
            table_hbm.at[pl.ds(0, CHUNK)], rows[slot], gsem[slot]).wait()

    def drain_store(slot):
        pltpu.make_async_copy(
            rows[slot], out_hbm.at[pl.ds(base, CHUNK)], ssem[slot]).wait()

    def chunk_ops(i, m4, drain_prev_store, start_next, prefetch_idx):
        s = m4 % 2
        nm4 = (m4 + 1) % 4
        if start_next:
            drain_idx(nm4)
        if drain_prev_store:
            drain_store(1 - s)
        if start_next:
            start_gather(i + 1, 1 - s, nm4)
        drain_gather(s)
        start_store(i, s)
        if prefetch_idx:
            start_idx(i + 4, m4)

    for m4 in range(4):
        start_idx(m4, m4)
    drain_idx(0)
    start_gather(0, 0, 0)
    for i in range(4):
        chunk_ops(i, i, drain_prev_store=(i >= 1), start_next=True,
                  prefetch_idx=True)

    @pl.loop(4, N_CHUNKS - 4, step=4)
    def _body(ib):
        for k in range(4):
            chunk_ops(ib + k, k, drain_prev_store=True, start_next=True,
                      prefetch_idx=True)

    for i in range(N_CHUNKS - 4, N_CHUNKS):
        chunk_ops(i, i % 4, drain_prev_store=True,
                  start_next=(i < N_CHUNKS - 1), prefetch_idx=False)
    drain_store((N_CHUNKS - 1) % 2)


@jax.jit
def kernel(indices, table):
    flat = indices.reshape(NW, N_CHUNKS, CHUNK).astype(jnp.int32)
    out = _gather_kernel(flat, table)
    return out.reshape(indices.shape + (EMBED,))


# speedup vs baseline: 1.0475x; 1.0004x over previous
import functools

import jax
import jax.numpy as jnp
from jax import lax
from jax.experimental import pallas as pl
from jax.experimental.pallas import tpu as pltpu
from jax.experimental.pallas import tpu_sc as plsc

B = 4096
L = 200
EMBED = 64
N_ROWS = B * L
NC = 2
NS = 16
NW = NC * NS
ROWS_PER_W = N_ROWS // NW
CHUNK = 640
N_CHUNKS = ROWS_PER_W // CHUNK
NSPLIT = 4
SUB = CHUNK // NSPLIT

_mesh = plsc.VectorSubcoreMesh(
    core_axis_name="c", subcore_axis_name="s", num_cores=NC, num_subcores=NS
)


@functools.partial(
    pl.kernel,
    out_type=jax.ShapeDtypeStruct((N_ROWS, EMBED), jnp.float32),
    mesh=_mesh,
    scratch_types=[
        pltpu.VMEM((N_CHUNKS, CHUNK), jnp.int32),
        pltpu.VMEM((CHUNK, EMBED), jnp.float32),
        pltpu.VMEM((CHUNK, EMBED), jnp.float32),
        pltpu.SemaphoreType.DMA,
        pltpu.SemaphoreType.DMA,
        pltpu.SemaphoreType.DMA,
        pltpu.SemaphoreType.DMA,
    ],
    compiler_params=pltpu.CompilerParams(use_tc_tiling_on_sc=False),
)
def _gather_kernel(idx_hbm, table_hbm, out_hbm, idx_v, rows0, rows1, g0, g1,
                   s0, s1):
    rows = (rows0, rows1)
    gsem = (g0, g1)
    ssem = (s0, s1)
    wid = lax.axis_index("s") * NC + lax.axis_index("c")
    base = wid * ROWS_PER_W

    def start_gather(i, slot):
        for h in range(NSPLIT):
            pltpu.async_copy(
                table_hbm.at[idx_v.at[i, pl.ds(h * SUB, SUB)]],
                rows[slot].at[pl.ds(h * SUB, SUB)],
                gsem[slot])

    def start_store(i, slot):
        pltpu.async_copy(
            rows[slot], out_hbm.at[pl.ds(base + i * CHUNK, CHUNK)],
            ssem[slot])

    def drain_gather(slot):
        pltpu.make_async_copy(
            table_hbm.at[pl.ds(0, CHUNK)], rows[slot], gsem[slot]).wait()

    def drain_store(slot):
        pltpu.make_async_copy(
            rows[slot], out_hbm.at[pl.ds(base, CHUNK)], ssem[slot]).wait()

    pltpu.sync_copy(idx_hbm.at[wid], idx_v)

    start_gather(0, 0)
    start_gather(1, 1)
    drain_gather(0)
    start_store(0, 0)

    @pl.loop(1, N_CHUNKS - 1, step=2)
    def _body(i0):
        for k in range(2):
            i = i0 + k
            s = (1, 0)[k]
            drain_store(1 - s)
            start_gather(i + 1, 1 - s)
            drain_gather(s)
            start_store(i, s)

    drain_gather(1)
    start_store(N_CHUNKS - 1, 1)
    drain_store(0)
    drain_store(1)


@jax.jit
def kernel(indices, table):
    flat = indices.reshape(NW, N_CHUNKS, CHUNK).astype(jnp.int32)
    out = _gather_kernel(flat, table)
    return out.reshape(indices.shape + (EMBED,))


# speedup vs baseline: 1.0497x; 1.0021x over previous
import functools

import jax
import jax.numpy as jnp
from jax import lax
from jax.experimental import pallas as pl
from jax.experimental.pallas import tpu as pltpu
from jax.experimental.pallas import tpu_sc as plsc

B = 4096
L = 200
EMBED = 64
NC = 2
NS = 16
NW = NC * NS
B_PER_W = B // NW
KB = 2
N_CHUNKS = B_PER_W // KB
CHUNK = KB * L

_mesh = plsc.VectorSubcoreMesh(
    core_axis_name="c", subcore_axis_name="s", num_cores=NC, num_subcores=NS
)


@functools.partial(
    pl.kernel,
    out_type=jax.ShapeDtypeStruct((B, L, EMBED), jnp.float32),
    mesh=_mesh,
    scratch_types=[
        pltpu.VMEM((B_PER_W, L), jnp.int32),
        pltpu.VMEM((KB, L, EMBED), jnp.float32),
        pltpu.VMEM((KB, L, EMBED), jnp.float32),
        pltpu.SemaphoreType.DMA,
        pltpu.SemaphoreType.DMA,
        pltpu.SemaphoreType.DMA,
        pltpu.SemaphoreType.DMA,
    ],
    compiler_params=pltpu.CompilerParams(use_tc_tiling_on_sc=False),
)
def _gather_kernel(idx_hbm, table_hbm, out_hbm, idx_v, rows0, rows1, g0, g1,
                   s0, s1):
    rows = (rows0, rows1)
    gsem = (g0, g1)
    ssem = (s0, s1)
    wid = lax.axis_index("s") * NC + lax.axis_index("c")
    base = wid * B_PER_W

    def start_gather(i, slot):
        for b in range(KB):
            pltpu.async_copy(
                table_hbm.at[idx_v.at[i * KB + b]],
                rows[slot].at[b],
                gsem[slot])

    def start_store(i, slot):
        pltpu.async_copy(
            rows[slot], out_hbm.at[pl.ds(base + i * KB, KB)], ssem[slot])

    def drain_gather(slot):
        pltpu.make_async_copy(
            rows[slot], out_hbm.at[pl.ds(base, KB)], gsem[slot]).wait()

    def drain_store(slot):
        pltpu.make_async_copy(
            rows[slot], out_hbm.at[pl.ds(base, KB)], ssem[slot]).wait()

    pltpu.sync_copy(idx_hbm.at[pl.ds(base, B_PER_W)], idx_v)

    start_gather(0, 0)
    start_gather(1, 1)
    drain_gather(0)
    start_store(0, 0)

    @pl.loop(1, N_CHUNKS - 1, step=2)
    def _body(i0):
        for k in range(2):
            i = i0 + k
            s = (1, 0)[k]
            drain_store(1 - s)
            start_gather(i + 1, 1 - s)
            drain_gather(s)
            start_store(i, s)

    drain_gather(1)
    start_store(N_CHUNKS - 1, 1)
    drain_store(0)
    drain_store(1)


@jax.jit
def kernel(indices, table):
    return _gather_kernel(indices.astype(jnp.int32), table)
